# baseline (speedup 1.0000x reference)
import functools

import jax
import jax.numpy as jnp
from jax import lax
from jax.experimental import pallas as pl
from jax.experimental.pallas import tpu as pltpu
from jax.experimental.pallas import tpu_sc as plsc

_NW = 32
_C = 128
_GBUF = 5
_OBUF = 2
_STEP = 10
_AHEAD = _GBUF - 1
_LANES = 16

_GDN = lax.GatherDimensionNumbers(
    offset_dims=(), collapsed_slice_dims=(0,), start_index_map=(0,))


def _bcast_lane(vec, lane):
    idx = jnp.full((_LANES, 1), lane, jnp.int32)
    return lax.gather(vec, idx, dimension_numbers=_GDN, slice_sizes=(1,),
                      mode=lax.GatherScatterMode.PROMISE_IN_BOUNDS)


def _sc_embed(xf, sf, token_table, strain_table, n, d):
    per_w = n // _NW
    n_chunks = per_w // _C
    n_outer = n_chunks // _STEP
    nj = d // _LANES
    mesh = plsc.VectorSubcoreMesh(core_axis_name="c", subcore_axis_name="s")

    @functools.partial(
        pl.kernel,
        mesh=mesh,
        out_type=jax.ShapeDtypeStruct((n, d), jnp.float32),
        scratch_types=(
            [pltpu.VMEM((n_chunks, _C), jnp.int32),
             pltpu.VMEM((n_chunks, _C), jnp.int32),
             pltpu.VMEM((3, d), jnp.float32)]
            + [pltpu.VMEM((_C, d), jnp.float32) for _ in range(_GBUF + _OBUF)]
            + [pltpu.SemaphoreType.DMA for _ in range(_GBUF + _OBUF)]
        ),
    )
    def k(xf_hbm, sf_hbm, tok_hbm, st_hbm, out_hbm,
          idx_v, sidx_v, stab_v, *bufs_and_sems):
        gbuf = bufs_and_sems[:_GBUF]
        obuf = bufs_and_sems[_GBUF:_GBUF + _OBUF]
        gsem = bufs_and_sems[_GBUF + _OBUF:2 * _GBUF + _OBUF]
        wsem = bufs_and_sems[2 * _GBUF + _OBUF:]
        wid = lax.axis_index("s") * 2 + lax.axis_index("c")
        wbase = wid * per_w

        pltpu.sync_copy(xf_hbm.at[wid], idx_v)
        pltpu.sync_copy(sf_hbm.at[wid], sidx_v)
        pltpu.sync_copy(st_hbm, stab_v)
        r0 = [stab_v[0, pl.ds(j * _LANES, _LANES)] for j in range(nj)]
        d10 = [stab_v[1, pl.ds(j * _LANES, _LANES)] - r0[j] for j in range(nj)]
        d21 = [stab_v[2, pl.ds(j * _LANES, _LANES)]
               - stab_v[1, pl.ds(j * _LANES, _LANES)] for j in range(nj)]

        def start_gather(ci, b):
            return pltpu.async_copy(tok_hbm.at[idx_v.at[ci]], gbuf[b], gsem[b])

        def wait_gather(ci, b):
            pltpu.make_async_copy(
                tok_hbm.at[idx_v.at[ci]], gbuf[b], gsem[b]).wait()

        def start_write(ci, b):
            return pltpu.async_copy(
                obuf[b], out_hbm.at[pl.ds(wbase + ci * _C, _C)], wsem[b])

        def wait_write(ci, b):
            pltpu.make_async_copy(
                obuf[b], out_hbm.at[pl.ds(wbase + ci * _C, _C)], wsem[b]).wait()

        def compute(ci, gb, ob):
            src = gbuf[gb]
            dst = obuf[ob]

            def quad(i, carry):
                sv16 = sidx_v[ci, pl.ds((i // 4) * _LANES, _LANES)]
                lane0 = (i % 4) * 4
                for rr in range(4):
                    row = i * 4 + rr
                    s_f = _bcast_lane(sv16, lane0 + rr).astype(jnp.float32)
                    f1 = jnp.minimum(s_f, 1.0)
                    f2 = jnp.maximum(s_f - 1.0, 0.0)
                    for j in range(nj):
                        sl = pl.ds(j * _LANES, _LANES)
                        st = r0[j] + f1 * d10[j] + f2 * d21[j]
                        dst[row, sl] = src[row, sl] + st
                return carry

            lax.fori_loop(0, _C // 4, quad, 0, unroll=False)

        for b in range(_AHEAD):
            start_gather(b, b)

        def outer(o, carry):
            for b in range(_STEP):
                g = o * _STEP + b
                gb = b % _GBUF
                ob = b % _OBUF

                wait_gather(g, gb)
                if b < _OBUF:
                    @pl.when(g >= _OBUF)
                    def _():
                        wait_write(g - _OBUF, ob)
                else:
                    wait_write(g - _OBUF, ob)
                compute(g, gb, ob)
                start_write(g, ob)

                f = g + _AHEAD

                @pl.when(f < n_chunks)
                def _():
                    start_gather(f, (b + _AHEAD) % _GBUF)
            return carry

        lax.fori_loop(0, n_outer, outer, 0, unroll=False)

        for b in range(_OBUF):
            wait_write(n_chunks - _OBUF + b, (n_chunks - _OBUF + b) % _OBUF)

    return k(xf, sf, token_table, strain_table)


def kernel(x, strains, token_table, strain_table):
    b, l = x.shape
    _, d = token_table.shape
    n = b * l
    per_w = n // _NW
    xw = x.reshape(_NW, per_w // _C, _C)
    sw = strains.reshape(_NW, per_w // _C, _C)
    out = _sc_embed(xw, sw, token_table, strain_table, n, d)
    return out.reshape(b, l, d)


# speedup vs baseline: 2.4656x; 2.4656x over previous
import functools

import jax
import jax.numpy as jnp
from jax import lax
from jax.experimental import pallas as pl
from jax.experimental.pallas import tpu as pltpu
from jax.experimental.pallas import tpu_sc as plsc

_NW = 32
_C = 128
_NBUF = 5
_LANES = 16

_GDN = lax.GatherDimensionNumbers(
    offset_dims=(), collapsed_slice_dims=(0,), start_index_map=(0,))


def _bcast_lane(vec, lane):
    idx = jnp.full((_LANES, 1), lane, jnp.int32)
    return lax.gather(vec, idx, dimension_numbers=_GDN, slice_sizes=(1,),
                      mode=lax.GatherScatterMode.PROMISE_IN_BOUNDS)


def _sc_embed(xf, sf, token_table, strain_table, n, d):
    per_w = n // _NW
    n_chunks = per_w // _C
    n_outer = n_chunks // _NBUF
    nj = d // _LANES
    mesh = plsc.VectorSubcoreMesh(core_axis_name="c", subcore_axis_name="s")

    @functools.partial(
        pl.kernel,
        mesh=mesh,
        out_type=jax.ShapeDtypeStruct((n, d), jnp.float32),
        scratch_types=(
            [pltpu.VMEM((n_chunks, _C), jnp.int32),
             pltpu.VMEM((n_chunks, _C), jnp.int32),
             pltpu.VMEM((3, d), jnp.float32)]
            + [pltpu.VMEM((_C, d), jnp.float32) for _ in range(_NBUF)]
            + [pltpu.SemaphoreType.DMA for _ in range(2 * _NBUF)]
        ),
    )
    def k(xf_hbm, sf_hbm, tok_hbm, st_hbm, out_hbm,
          idx_v, sidx_v, stab_v, *bufs_and_sems):
        bufs = bufs_and_sems[:_NBUF]
        gsem = bufs_and_sems[_NBUF:2 * _NBUF]
        wsem = bufs_and_sems[2 * _NBUF:]
        wid = lax.axis_index("s") * 2 + lax.axis_index("c")
        wbase = wid * per_w

        pltpu.sync_copy(xf_hbm.at[wid], idx_v)
        pltpu.sync_copy(sf_hbm.at[wid], sidx_v)
        pltpu.sync_copy(st_hbm, stab_v)
        r0 = [stab_v[0, pl.ds(j * _LANES, _LANES)] for j in range(nj)]
        d10 = [stab_v[1, pl.ds(j * _LANES, _LANES)] - r0[j] for j in range(nj)]
        d21 = [stab_v[2, pl.ds(j * _LANES, _LANES)]
               - stab_v[1, pl.ds(j * _LANES, _LANES)] for j in range(nj)]

        def start_gather(ci, b):
            return pltpu.async_copy(tok_hbm.at[idx_v.at[ci]], bufs[b], gsem[b])

        def wait_gather(ci, b):
            pltpu.make_async_copy(
                tok_hbm.at[idx_v.at[ci]], bufs[b], gsem[b]).wait()

        def start_write(ci, b):
            return pltpu.async_copy(
                bufs[b], out_hbm.at[pl.ds(wbase + ci * _C, _C)], wsem[b])

        def wait_write(ci, b):
            pltpu.make_async_copy(
                bufs[b], out_hbm.at[pl.ds(wbase + ci * _C, _C)], wsem[b]).wait()

        def compute(ci, b):
            buf = bufs[b]

            def quad(i, carry):
                sv16 = sidx_v[ci, pl.ds((i // 4) * _LANES, _LANES)]
                s_f = sv16.astype(jnp.float32)
                f1g = jnp.minimum(s_f, 1.0)
                f2g = jnp.maximum(s_f - 1.0, 0.0)
                lane0 = (i % 4) * 4
                for rr in range(4):
                    row = i * 4 + rr
                    f1 = _bcast_lane(f1g, lane0 + rr)
                    f2 = _bcast_lane(f2g, lane0 + rr)
                    for j in range(nj):
                        sl = pl.ds(j * _LANES, _LANES)
                        st = r0[j] + f1 * d10[j] + f2 * d21[j]
                        plsc.addupdate(buf.at[row, sl], st)
                return carry

            lax.fori_loop(0, _C // 4, quad, 0, unroll=False)

        for b in range(_NBUF - 1):
            start_gather(b, b)

        for b in range(_NBUF):
            f = b + _NBUF - 1
            pb = f % _NBUF
            if f >= _NBUF:
                wait_write(b - 1, pb)
            start_gather(f, pb)
            wait_gather(b, b)
            compute(b, b)
            start_write(b, b)

        def outer(o, carry):
            for b in range(_NBUF):
                g = o * _NBUF + b
                f = g + _NBUF - 1
                pb = (b + _NBUF - 1) % _NBUF

                @pl.when(f < n_chunks)
                def _():
                    wait_write(f - _NBUF, pb)
                    start_gather(f, pb)

                wait_gather(g, b)
                compute(g, b)
                start_write(g, b)
            return carry

        lax.fori_loop(1, n_outer, outer, 0, unroll=False)

        for b in range(_NBUF):
            wait_write(n_chunks - _NBUF + b, b)

    return k(xf, sf, token_table, strain_table)


def kernel(x, strains, token_table, strain_table):
    b, l = x.shape
    _, d = token_table.shape
    n = b * l
    per_w = n // _NW
    xw = x.reshape(_NW, per_w // _C, _C)
    sw = strains.reshape(_NW, per_w // _C, _C)
    out = _sc_embed(xw, sw, token_table, strain_table, n, d)
    return out.reshape(b, l, d)


# speedup vs baseline: 2.7792x; 1.1272x over previous
import functools

import jax
import jax.numpy as jnp
from jax import lax
from jax.experimental import pallas as pl
from jax.experimental.pallas import tpu as pltpu
from jax.experimental.pallas import tpu_sc as plsc

_NW = 32
_C = 128
_NBUF = 5
_LANES = 16

_GDN = lax.GatherDimensionNumbers(
    offset_dims=(), collapsed_slice_dims=(0,), start_index_map=(0,))


def _bcast_lane(vec, lane):
    idx = jnp.full((_LANES, 1), lane, jnp.int32)
    return lax.gather(vec, idx, dimension_numbers=_GDN, slice_sizes=(1,),
                      mode=lax.GatherScatterMode.PROMISE_IN_BOUNDS)


def _sc_embed(xf, sf, token_table, strain_table, n, d):
    per_w = n // _NW
    n_chunks = per_w // _C
    n_outer = n_chunks // _NBUF
    nj = d // _LANES
    mesh = plsc.VectorSubcoreMesh(core_axis_name="c", subcore_axis_name="s")

    @functools.partial(
        pl.kernel,
        mesh=mesh,
        out_type=jax.ShapeDtypeStruct((n, d), jnp.float32),
        scratch_types=(
            [pltpu.VMEM((n_chunks, _C), jnp.int32),
             pltpu.VMEM((n_chunks, _C), jnp.int32),
             pltpu.VMEM((3, d), jnp.float32)]
            + [pltpu.VMEM((_C, d), jnp.float32) for _ in range(_NBUF)]
            + [pltpu.SemaphoreType.DMA for _ in range(2 * _NBUF)]
        ),
    )
    def k(xf_hbm, sf_hbm, tok_hbm, st_hbm, out_hbm,
          idx_v, sidx_v, stab_v, *bufs_and_sems):
        bufs = bufs_and_sems[:_NBUF]
        gsem = bufs_and_sems[_NBUF:2 * _NBUF]
        wsem = bufs_and_sems[2 * _NBUF:]
        wid = lax.axis_index("s") * 2 + lax.axis_index("c")
        wbase = wid * per_w

        pltpu.sync_copy(xf_hbm.at[wid], idx_v)
        pltpu.sync_copy(sf_hbm.at[wid], sidx_v)
        pltpu.sync_copy(st_hbm, stab_v)
        r0 = [stab_v[0, pl.ds(j * _LANES, _LANES)] for j in range(nj)]
        d10 = [stab_v[1, pl.ds(j * _LANES, _LANES)] - r0[j] for j in range(nj)]
        d21 = [stab_v[2, pl.ds(j * _LANES, _LANES)]
               - stab_v[1, pl.ds(j * _LANES, _LANES)] for j in range(nj)]

        def start_gather(ci, b):
            return pltpu.async_copy(tok_hbm.at[idx_v.at[ci]], bufs[b], gsem[b])

        def wait_gather(ci, b):
            pltpu.make_async_copy(
                tok_hbm.at[idx_v.at[ci]], bufs[b], gsem[b]).wait()

        half = _C // 2

        def start_write_half(ci, b, h):
            return pltpu.async_copy(
                bufs[b].at[pl.ds(h * half, half)],
                out_hbm.at[pl.ds(wbase + ci * _C + h * half, half)], wsem[b])

        def wait_write(ci, b):
            for h in range(2):
                pltpu.make_async_copy(
                    bufs[b].at[pl.ds(h * half, half)],
                    out_hbm.at[pl.ds(wbase + ci * _C + h * half, half)],
                    wsem[b]).wait()

        def compute_half(ci, b, h):
            buf = bufs[b]

            def quad(i, carry):
                sv16 = sidx_v[ci, pl.ds((i // 4) * _LANES, _LANES)]
                s_f = sv16.astype(jnp.float32)
                f1g = jnp.minimum(s_f, 1.0)
                f2g = jnp.maximum(s_f - 1.0, 0.0)
                lane0 = (i % 4) * 4
                for rr in range(4):
                    row = i * 4 + rr
                    f1 = _bcast_lane(f1g, lane0 + rr)
                    f2 = _bcast_lane(f2g, lane0 + rr)
                    for j in range(nj):
                        sl = pl.ds(j * _LANES, _LANES)
                        st = r0[j] + f1 * d10[j] + f2 * d21[j]
                        plsc.addupdate(buf.at[row, sl], st)
                return carry

            lax.fori_loop(h * (half // 4), (h + 1) * (half // 4), quad, 0,
                          unroll=False)

        def compute_and_write(ci, b):
            for h in range(2):
                compute_half(ci, b, h)
                start_write_half(ci, b, h)

        for b in range(_NBUF - 1):
            start_gather(b, b)

        for b in range(_NBUF):
            f = b + _NBUF - 1
            pb = f % _NBUF
            wait_gather(b, b)
            compute_and_write(b, b)
            if f >= _NBUF:
                wait_write(b - 1, pb)
            start_gather(f, pb)

        def outer(o, carry):
            for b in range(_NBUF):
                g = o * _NBUF + b
                f = g + _NBUF - 1
                pb = (b + _NBUF - 1) % _NBUF

                @pl.when(f < n_chunks)
                def _():
                    wait_write(f - _NBUF, pb)
                    start_gather(f, pb)

                wait_gather(g, b)
                compute_and_write(g, b)
            return carry

        lax.fori_loop(1, n_outer, outer, 0, unroll=False)

        for b in range(_NBUF):
            wait_write(n_chunks - _NBUF + b, b)

    return k(xf, sf, token_table, strain_table)


def kernel(x, strains, token_table, strain_table):
    b, l = x.shape
    _, d = token_table.shape
    n = b * l
    per_w = n // _NW
    xw = x.reshape(_NW, per_w // _C, _C)
    sw = strains.reshape(_NW, per_w // _C, _C)
    out = _sc_embed(xw, sw, token_table, strain_table, n, d)
    return out.reshape(b, l, d)


# speedup vs baseline: 2.7842x; 1.0018x over previous
import functools

import jax
import jax.numpy as jnp
from jax import lax
from jax.experimental import pallas as pl
from jax.experimental.pallas import tpu as pltpu
from jax.experimental.pallas import tpu_sc as plsc

_NW = 32
_C = 128
_NBUF = 5
_LANES = 16

_GDN = lax.GatherDimensionNumbers(
    offset_dims=(), collapsed_slice_dims=(0,), start_index_map=(0,))


def _bcast_lane(vec, lane):
    idx = jnp.full((_LANES, 1), lane, jnp.int32)
    return lax.gather(vec, idx, dimension_numbers=_GDN, slice_sizes=(1,),
                      mode=lax.GatherScatterMode.PROMISE_IN_BOUNDS)


def _sc_embed(xf, sf, token_table, strain_table, n, d):
    per_w = n // _NW
    n_chunks = per_w // _C
    n_outer = n_chunks // _NBUF
    nj = d // _LANES
    mesh = plsc.VectorSubcoreMesh(core_axis_name="c", subcore_axis_name="s")

    @functools.partial(
        pl.kernel,
        mesh=mesh,
        out_type=jax.ShapeDtypeStruct((n, d), jnp.float32),
        scratch_types=(
            [pltpu.VMEM((n_chunks, _C), jnp.int32),
             pltpu.VMEM((n_chunks, _C), jnp.int32),
             pltpu.VMEM((3, d), jnp.float32)]
            + [pltpu.VMEM((_C, d), jnp.float32) for _ in range(_NBUF)]
            + [pltpu.SemaphoreType.DMA for _ in range(2 * _NBUF)]
        ),
    )
    def k(xf_hbm, sf_hbm, tok_hbm, st_hbm, out_hbm,
          idx_v, sidx_v, stab_v, *bufs_and_sems):
        bufs = bufs_and_sems[:_NBUF]
        gsem = bufs_and_sems[_NBUF:2 * _NBUF]
        wsem = bufs_and_sems[2 * _NBUF:]
        wid = lax.axis_index("s") * 2 + lax.axis_index("c")
        wbase = wid * per_w

        pltpu.sync_copy(xf_hbm.at[wid], idx_v)
        pltpu.sync_copy(sf_hbm.at[wid], sidx_v)
        pltpu.sync_copy(st_hbm, stab_v)
        r0 = [stab_v[0, pl.ds(j * _LANES, _LANES)] for j in range(nj)]
        d10 = [stab_v[1, pl.ds(j * _LANES, _LANES)] - r0[j] for j in range(nj)]
        d21 = [stab_v[2, pl.ds(j * _LANES, _LANES)]
               - stab_v[1, pl.ds(j * _LANES, _LANES)] for j in range(nj)]

        def start_gather(ci, b):
            return pltpu.async_copy(tok_hbm.at[idx_v.at[ci]], bufs[b], gsem[b])

        def wait_gather(ci, b):
            pltpu.make_async_copy(
                tok_hbm.at[idx_v.at[ci]], bufs[b], gsem[b]).wait()

        half = _C // 2

        def start_write_half(ci, b, h):
            return pltpu.async_copy(
                bufs[b].at[pl.ds(h * half, half)],
                out_hbm.at[pl.ds(wbase + ci * _C + h * half, half)], wsem[b])

        def wait_write(ci, b):
            for h in range(2):
                pltpu.make_async_copy(
                    bufs[b].at[pl.ds(h * half, half)],
                    out_hbm.at[pl.ds(wbase + ci * _C + h * half, half)],
                    wsem[b]).wait()

        def compute_half(ci, b, h):
            buf = bufs[b]

            def quad(i, carry):
                sv16 = sidx_v[ci, pl.ds((i // 4) * _LANES, _LANES)]
                s_f = sv16.astype(jnp.float32)
                f1g = jnp.minimum(s_f, 1.0)
                f2g = jnp.maximum(s_f - 1.0, 0.0)
                lane0 = (i % 4) * 4
                for rr in range(4):
                    row = i * 4 + rr
                    f1 = _bcast_lane(f1g, lane0 + rr)
                    f2 = _bcast_lane(f2g, lane0 + rr)
                    for j in range(nj):
                        sl = pl.ds(j * _LANES, _LANES)
                        st = r0[j] + f1 * d10[j] + f2 * d21[j]
                        plsc.addupdate(buf.at[row, sl], st)
                return carry

            lax.fori_loop(h * (half // 4), (h + 1) * (half // 4), quad, 0,
                          unroll=False)

        def compute_and_write(ci, b, between=None):
            compute_half(ci, b, 0)
            start_write_half(ci, b, 0)
            if between is not None:
                between()
            compute_half(ci, b, 1)
            start_write_half(ci, b, 1)

        for b in range(_NBUF - 1):
            start_gather(b, b)

        for b in range(_NBUF):
            f = b + _NBUF - 1
            pb = f % _NBUF
            def warm_prefetch(b=b, f=f, pb=pb):
                if f >= _NBUF:
                    wait_write(b - 1, pb)
                start_gather(f, pb)

            wait_gather(b, b)
            compute_and_write(b, b, between=warm_prefetch)

        def outer(o, carry):
            for b in range(_NBUF):
                g = o * _NBUF + b
                f = g + _NBUF - 1
                pb = (b + _NBUF - 1) % _NBUF

                def prefetch(f=f, pb=pb):
                    @pl.when(f < n_chunks)
                    def _():
                        wait_write(f - _NBUF, pb)
                        start_gather(f, pb)

                wait_gather(g, b)
                compute_and_write(g, b, between=prefetch)
            return carry

        lax.fori_loop(1, n_outer, outer, 0, unroll=False)

        for b in range(_NBUF):
            wait_write(n_chunks - _NBUF + b, b)

    return k(xf, sf, token_table, strain_table)


def kernel(x, strains, token_table, strain_table):
    b, l = x.shape
    _, d = token_table.shape
    n = b * l
    per_w = n // _NW
    xw = x.reshape(_NW, per_w // _C, _C)
    sw = strains.reshape(_NW, per_w // _C, _C)
    out = _sc_embed(xw, sw, token_table, strain_table, n, d)
    return out.reshape(b, l, d)


# speedup vs baseline: 2.8279x; 1.0157x over previous
import functools

import jax
import jax.numpy as jnp
from jax import lax
from jax.experimental import pallas as pl
from jax.experimental.pallas import tpu as pltpu
from jax.experimental.pallas import tpu_sc as plsc

_NW = 32
_C = 128
_NBUF = 5
_LANES = 16

_GDN = lax.GatherDimensionNumbers(
    offset_dims=(), collapsed_slice_dims=(0,), start_index_map=(0,))


def _bcast_lane(vec, lane):
    idx = jnp.full((_LANES, 1), lane, jnp.int32)
    return lax.gather(vec, idx, dimension_numbers=_GDN, slice_sizes=(1,),
                      mode=lax.GatherScatterMode.PROMISE_IN_BOUNDS)


def _sc_embed(xf, sf, token_table, strain_table, n, d):
    per_w = n // _NW
    n_chunks = per_w // _C
    n_outer = n_chunks // _NBUF
    nj = d // _LANES
    mesh = plsc.VectorSubcoreMesh(core_axis_name="c", subcore_axis_name="s")

    @functools.partial(
        pl.kernel,
        mesh=mesh,
        out_type=jax.ShapeDtypeStruct((n, d), jnp.float32),
        scratch_types=(
            [pltpu.VMEM((n_chunks, _C), jnp.int32),
             pltpu.VMEM((n_chunks, _C), jnp.int32),
             pltpu.VMEM((3, d), jnp.float32)]
            + [pltpu.VMEM((_C, d), jnp.float32) for _ in range(_NBUF)]
            + [pltpu.SemaphoreType.DMA for _ in range(2 * _NBUF + 3)]
        ),
    )
    def k(xf_hbm, sf_hbm, tok_hbm, st_hbm, out_hbm,
          idx_v, sidx_v, stab_v, *bufs_and_sems):
        bufs = bufs_and_sems[:_NBUF]
        gsem = bufs_and_sems[_NBUF:2 * _NBUF]
        wsem = bufs_and_sems[2 * _NBUF:3 * _NBUF]
        isem, ssem, tsem = bufs_and_sems[3 * _NBUF:]
        wid = lax.axis_index("s") * 2 + lax.axis_index("c")
        wbase = wid * per_w

        icp = pltpu.async_copy(xf_hbm.at[wid], idx_v, isem)
        scp = pltpu.async_copy(sf_hbm.at[wid], sidx_v, ssem)
        tcp = pltpu.async_copy(st_hbm, stab_v, tsem)
        icp.wait()

        def start_gather(ci, b):
            return pltpu.async_copy(tok_hbm.at[idx_v.at[ci]], bufs[b], gsem[b])

        def wait_gather(ci, b):
            pltpu.make_async_copy(
                tok_hbm.at[idx_v.at[ci]], bufs[b], gsem[b]).wait()

        half = _C // 2

        def start_write_half(ci, b, h):
            return pltpu.async_copy(
                bufs[b].at[pl.ds(h * half, half)],
                out_hbm.at[pl.ds(wbase + ci * _C + h * half, half)], wsem[b])

        def wait_write(ci, b):
            for h in range(2):
                pltpu.make_async_copy(
                    bufs[b].at[pl.ds(h * half, half)],
                    out_hbm.at[pl.ds(wbase + ci * _C + h * half, half)],
                    wsem[b]).wait()

        def compute_half(ci, b, h):
            buf = bufs[b]

            def quad(i, carry):
                sv16 = sidx_v[ci, pl.ds((i // 4) * _LANES, _LANES)]
                s_f = sv16.astype(jnp.float32)
                f1g = jnp.minimum(s_f, 1.0)
                f2g = jnp.maximum(s_f - 1.0, 0.0)
                lane0 = (i % 4) * 4
                for rr in range(4):
                    row = i * 4 + rr
                    f1 = _bcast_lane(f1g, lane0 + rr)
                    f2 = _bcast_lane(f2g, lane0 + rr)
                    for j in range(nj):
                        sl = pl.ds(j * _LANES, _LANES)
                        st = r0[j] + f1 * d10[j] + f2 * d21[j]
                        plsc.addupdate(buf.at[row, sl], st)
                return carry

            lax.fori_loop(h * (half // 4), (h + 1) * (half // 4), quad, 0,
                          unroll=False)

        def compute_and_write(ci, b, between=None):
            compute_half(ci, b, 0)
            start_write_half(ci, b, 0)
            if between is not None:
                between()
            compute_half(ci, b, 1)
            start_write_half(ci, b, 1)

        for b in range(_NBUF - 1):
            start_gather(b, b)

        tcp.wait()
        scp.wait()
        r0 = [stab_v[0, pl.ds(j * _LANES, _LANES)] for j in range(nj)]
        d10 = [stab_v[1, pl.ds(j * _LANES, _LANES)] - r0[j] for j in range(nj)]
        d21 = [stab_v[2, pl.ds(j * _LANES, _LANES)]
               - stab_v[1, pl.ds(j * _LANES, _LANES)] for j in range(nj)]

        for b in range(_NBUF):
            f = b + _NBUF - 1
            pb = f % _NBUF
            def warm_prefetch(b=b, f=f, pb=pb):
                if f >= _NBUF:
                    wait_write(b - 1, pb)
                start_gather(f, pb)

            wait_gather(b, b)
            compute_and_write(b, b, between=warm_prefetch)

        def outer(o, carry):
            for b in range(_NBUF):
                g = o * _NBUF + b
                f = g + _NBUF - 1
                pb = (b + _NBUF - 1) % _NBUF

                def prefetch(f=f, pb=pb):
                    @pl.when(f < n_chunks)
                    def _():
                        wait_write(f - _NBUF, pb)
                        start_gather(f, pb)

                wait_gather(g, b)
                compute_and_write(g, b, between=prefetch)
            return carry

        lax.fori_loop(1, n_outer, outer, 0, unroll=False)

        for b in range(_NBUF):
            wait_write(n_chunks - _NBUF + b, b)

    return k(xf, sf, token_table, strain_table)


def kernel(x, strains, token_table, strain_table):
    b, l = x.shape
    _, d = token_table.shape
    n = b * l
    per_w = n // _NW
    xw = x.reshape(_NW, per_w // _C, _C)
    sw = strains.reshape(_NW, per_w // _C, _C)
    out = _sc_embed(xw, sw, token_table, strain_table, n, d)
    return out.reshape(b, l, d)
